# Initial kernel scaffold; baseline (speedup 1.0000x reference)
import functools

import jax
import jax.numpy as jnp
from jax import lax
from jax.experimental import pallas as pl
from jax.experimental.pallas import tpu as pltpu
from jax.experimental.pallas import tpu_sc as plsc

N = 10000
E = 160000
D = 256
NG = 64

NW = 32
NPT = 313
NP = NW * NPT
SCHUNK = 4000
NCH = E // SCHUNK
CAP = 1024
LSZ = 1088
GB = 64

BE = 1280
BN = 1000


def _edge_mm_body(ea_ref, w_ref, b_ref, out_ref):
    out_ref[...] = (
        jnp.dot(ea_ref[...], w_ref[...], preferred_element_type=jnp.float32)
        + b_ref[...]
    )


def _edge_mm(edge_attr, W1b, b1):
    return pl.pallas_call(
        _edge_mm_body,
        grid=(E // BE,),
        in_specs=[
            pl.BlockSpec((BE, D), lambda i: (i, 0)),
            pl.BlockSpec((D, D), lambda i: (0, 0)),
            pl.BlockSpec((1, D), lambda i: (0, 0)),
        ],
        out_specs=pl.BlockSpec((BE, D), lambda i: (i, 0)),
        out_shape=jax.ShapeDtypeStruct((E, D), jnp.float32),
    )(edge_attr, W1b, b1)


def _node_pre_body(x_ref, u_ref, w1a_ref, w2b_ref, w2c_ref, b2_ref, w4b_ref,
                   b4_ref, xw1_ref, xup_ref, uw4_ref):
    x = x_ref[...]
    u = u_ref[...]
    xw1_ref[...] = jnp.dot(x, w1a_ref[...], preferred_element_type=jnp.float32)
    xup_ref[...] = (
        jnp.dot(x, w2b_ref[...], preferred_element_type=jnp.float32)
        + jnp.dot(u, w2c_ref[...], preferred_element_type=jnp.float32)
        + b2_ref[...]
    )
    uw4_ref[...] = (
        jnp.dot(u, w4b_ref[...], preferred_element_type=jnp.float32)
        + b4_ref[...]
    )


def _node_pre(x, u, W1a, W2b, W2c, b2, W4b, b4):
    return pl.pallas_call(
        _node_pre_body,
        grid=(N // BN,),
        in_specs=[
            pl.BlockSpec((BN, D), lambda i: (i, 0)),
            pl.BlockSpec((BN, D), lambda i: (i, 0)),
            pl.BlockSpec((D, D), lambda i: (0, 0)),
            pl.BlockSpec((D, 4 * D), lambda i: (0, 0)),
            pl.BlockSpec((D, 4 * D), lambda i: (0, 0)),
            pl.BlockSpec((1, 4 * D), lambda i: (0, 0)),
            pl.BlockSpec((D, D), lambda i: (0, 0)),
            pl.BlockSpec((1, D), lambda i: (0, 0)),
        ],
        out_specs=[
            pl.BlockSpec((BN, D), lambda i: (i, 0)),
            pl.BlockSpec((BN, 4 * D), lambda i: (i, 0)),
            pl.BlockSpec((BN, D), lambda i: (i, 0)),
        ],
        out_shape=[
            jax.ShapeDtypeStruct((N, D), jnp.float32),
            jax.ShapeDtypeStruct((N, 4 * D), jnp.float32),
            jax.ShapeDtypeStruct((N, D), jnp.float32),
        ],
    )(x, u, W1a, W2b, W2c, b2, W4b, b4)


def _sc_agg_body(col_hbm, row_hbm, xw_hbm, ea_hbm, agg_hbm,
                 colbuf, rowbuf, eidl, rowl, lcoll, xg, eg, acc, nm_ref,
                 sem1, sem2):
    wid = lax.axis_index("s") * 2 + lax.axis_index("c")
    lo = wid * NPT
    hi = lo + NPT
    zero16 = jnp.zeros((16,), jnp.float32)
    zero16i = jnp.zeros((16,), jnp.int32)
    iota16 = lax.iota(jnp.int32, 16)

    def _z_acc(t, _):
        acc[pl.ds(t * 16, 16)] = zero16
        return 0
    lax.fori_loop(0, (NPT * D) // 16, _z_acc, 0)

    def _z_lists(t, _):
        eidl[pl.ds(t * 16, 16)] = zero16i
        rowl[pl.ds(t * 16, 16)] = zero16i
        return 0
    lax.fori_loop(0, LSZ // 16, _z_lists, 0)

    nm_ref[0] = 0

    def _flush():
        n = nm_ref[0]
        nit = (n + (GB - 1)) // GB

        def _gather_batch(k, _):
            off = k * GB
            cp1 = pltpu.async_copy(xw_hbm.at[rowl.at[pl.ds(off, GB)]], xg, sem1)
            cp2 = pltpu.async_copy(ea_hbm.at[eidl.at[pl.ds(off, GB)]], eg, sem2)
            cp1.wait()
            cp2.wait()

            def _row(r, _):
                @pl.when(off + r < n)
                def _():
                    lc = lcoll[off + r]
                    base = lc * D
                    for j in range(D // 16):
                        val = xg[r, pl.ds(16 * j, 16)] + eg[r, pl.ds(16 * j, 16)]
                        cur = acc[pl.ds(base + 16 * j, 16)]
                        acc[pl.ds(base + 16 * j, 16)] = jnp.maximum(cur, val)
                return 0

            lax.fori_loop(0, GB, _row, 0)
            return 0

        lax.fori_loop(0, nit, _gather_batch, 0)
        nm_ref[0] = 0

    def _chunk(c, _):
        pltpu.sync_copy(col_hbm.at[pl.ds(c * SCHUNK, SCHUNK)], colbuf)
        pltpu.sync_copy(row_hbm.at[pl.ds(c * SCHUNK, SCHUNK)], rowbuf)

        def _scan(t, _):
            v = colbuf[pl.ds(t * 16, 16)]
            r = rowbuf[pl.ds(t * 16, 16)]
            m = (v >= lo) & (v < hi)
            cnt = jnp.sum(m.astype(jnp.int32))
            nm = nm_ref[0]

            @pl.when(cnt > 0)
            def _():
                eids = c * SCHUNK + t * 16 + iota16
                plsc.store_compressed(lcoll.at[pl.ds(nm, 16)], v - lo, mask=m)
                plsc.store_compressed(rowl.at[pl.ds(nm, 16)], r, mask=m)
                plsc.store_compressed(eidl.at[pl.ds(nm, 16)], eids, mask=m)

            nm_ref[0] = nm + cnt

            @pl.when(nm + cnt >= CAP)
            def _():
                _flush()
            return 0

        lax.fori_loop(0, SCHUNK // 16, _scan, 0)
        return 0

    lax.fori_loop(0, NCH, _chunk, 0)
    _flush()

    pltpu.sync_copy(acc, agg_hbm.at[pl.ds(wid * NPT * D, NPT * D)])


def _sc_agg(col, row, xW1, eaW):
    mesh = plsc.VectorSubcoreMesh(core_axis_name="c", subcore_axis_name="s")
    f = functools.partial(
        pl.kernel,
        mesh=mesh,
        out_type=jax.ShapeDtypeStruct((NP * D,), jnp.float32),
        scratch_types=[
            pltpu.VMEM((SCHUNK,), jnp.int32),
            pltpu.VMEM((SCHUNK,), jnp.int32),
            pltpu.VMEM((LSZ,), jnp.int32),
            pltpu.VMEM((LSZ,), jnp.int32),
            pltpu.VMEM((LSZ,), jnp.int32),
            pltpu.VMEM((GB, D), jnp.float32),
            pltpu.VMEM((GB, D), jnp.float32),
            pltpu.VMEM((NPT * D,), jnp.float32),
            pltpu.SMEM((1,), jnp.int32),
            pltpu.SemaphoreType.DMA,
            pltpu.SemaphoreType.DMA,
        ],
    )(_sc_agg_body)
    return f(col, row, xW1, eaW)


def _node_mlp_body(agg_ref, xup_ref, x_ref, batchv_ref, batchs_ref,
                   w2a_ref, w3_ref, b3_ref, x2_ref, sraw_ref, acc_ref):
    i = pl.program_id(0)
    neg = jnp.float32(-jnp.inf)

    @pl.when(i == 0)
    def _():
        acc_ref[...] = jnp.full((NG, D), neg, jnp.float32)

    r1 = jax.nn.relu(
        jnp.dot(agg_ref[...], w2a_ref[...], preferred_element_type=jnp.float32)
        + xup_ref[...]
    )
    h = jax.nn.sigmoid(
        jnp.dot(r1, w3_ref[...], preferred_element_type=jnp.float32)
        + b3_ref[...]
    )
    x2 = x_ref[...] + h
    x2_ref[...] = x2

    bv = batchv_ref[...]
    g_lo = batchs_ref[0]
    g_hi = batchs_ref[BN - 1]

    def _g(g, _):
        msk = bv == g
        m = jnp.max(jnp.where(msk, x2, neg), axis=0, keepdims=True)
        acc_ref[pl.ds(g, 1), :] = jnp.maximum(acc_ref[pl.ds(g, 1), :], m)
        return 0

    lax.fori_loop(g_lo, g_hi + 1, _g, 0, unroll=False)
    sraw_ref[...] = acc_ref[...]


def _node_mlp(agg, xup, x, batch2d, batch, W2a, W3, b3):
    return pl.pallas_call(
        _node_mlp_body,
        grid=(N // BN,),
        in_specs=[
            pl.BlockSpec((BN, D), lambda i: (i, 0)),
            pl.BlockSpec((BN, 4 * D), lambda i: (i, 0)),
            pl.BlockSpec((BN, D), lambda i: (i, 0)),
            pl.BlockSpec((BN, 1), lambda i: (i, 0)),
            pl.BlockSpec((BN,), lambda i: (i,), memory_space=pltpu.SMEM),
            pl.BlockSpec((D, 4 * D), lambda i: (0, 0)),
            pl.BlockSpec((4 * D, D), lambda i: (0, 0)),
            pl.BlockSpec((1, D), lambda i: (0, 0)),
        ],
        out_specs=[
            pl.BlockSpec((BN, D), lambda i: (i, 0)),
            pl.BlockSpec((NG, D), lambda i: (0, 0)),
        ],
        out_shape=[
            jax.ShapeDtypeStruct((N, D), jnp.float32),
            jax.ShapeDtypeStruct((NG, D), jnp.float32),
        ],
        scratch_shapes=[pltpu.VMEM((NG, D), jnp.float32)],
    )(agg, xup, x, batch2d, batch, W2a, W3, b3)


def _glob_body(u_ref, uw4_ref, sraw_ref, w4a_ref, u2_ref):
    i = pl.program_id(0)
    s = sraw_ref[...]
    s = jnp.where(jnp.isinf(s), 0.0, s)
    t = jnp.dot(s, w4a_ref[...], preferred_element_type=jnp.float32)
    pad = jnp.concatenate([t, jnp.zeros((BN - NG, D), jnp.float32)], axis=0)
    addend = jnp.where(i == 0, pad, jnp.zeros_like(pad))
    u2_ref[...] = u_ref[...] + jax.nn.relu(uw4_ref[...] + addend)


def _glob(u, uw4, sraw, W4a):
    return pl.pallas_call(
        _glob_body,
        grid=(N // BN,),
        in_specs=[
            pl.BlockSpec((BN, D), lambda i: (i, 0)),
            pl.BlockSpec((BN, D), lambda i: (i, 0)),
            pl.BlockSpec((NG, D), lambda i: (0, 0)),
            pl.BlockSpec((D, D), lambda i: (0, 0)),
        ],
        out_specs=pl.BlockSpec((BN, D), lambda i: (i, 0)),
        out_shape=jax.ShapeDtypeStruct((N, D), jnp.float32),
    )(u, uw4, sraw, W4a)


def kernel(x, edge_index, edge_attr, u, batch, W1, b1, W2, b2, W3, b3, W4, b4):
    row = edge_index[0].astype(jnp.int32)
    col = edge_index[1].astype(jnp.int32)
    batch_i = batch.astype(jnp.int32)

    W1a = W1[:D]
    W1b = W1[D:]
    W2a = W2[:D]
    W2b = W2[D : 2 * D]
    W2c = W2[2 * D :]
    W4a = W4[:D]
    W4b = W4[D:]

    eaW = _edge_mm(edge_attr, W1b, b1.reshape(1, D))
    xW1, xup, uw4 = _node_pre(
        x, u, W1a, W2b, W2c, b2.reshape(1, 4 * D), W4b, b4.reshape(1, D)
    )

    agg_flat = _sc_agg(col, row, xW1, eaW)
    agg = agg_flat.reshape(NP, D)[:N]

    x2, sraw = _node_mlp(
        agg, xup, x, batch_i.reshape(N, 1), batch_i, W2a, W3, b3.reshape(1, D)
    )
    u2 = _glob(u, uw4, sraw, W4a)

    return (x2, edge_index, edge_attr, u2, batch)


# speedup vs baseline: 1.1778x; 1.1778x over previous
import functools

import jax
import jax.numpy as jnp
from jax import lax
from jax.experimental import pallas as pl
from jax.experimental.pallas import tpu as pltpu
from jax.experimental.pallas import tpu_sc as plsc

N = 10000
E = 160000
D = 256
NG = 64

NW = 32
NPT = 313
NP = NW * NPT
SCHUNK = 4000
NCH = E // SCHUNK
CAP = 1024
LSZ = 1088
GB = 64

BE = 1280
BN = 1000


def _edge_mm_body(ea_ref, w_ref, b_ref, out_ref):
    out_ref[...] = (
        jnp.dot(ea_ref[...], w_ref[...], preferred_element_type=jnp.float32)
        + b_ref[...]
    )


def _edge_mm(edge_attr, W1b, b1):
    return pl.pallas_call(
        _edge_mm_body,
        grid=(E // BE,),
        in_specs=[
            pl.BlockSpec((BE, D), lambda i: (i, 0)),
            pl.BlockSpec((D, D), lambda i: (0, 0)),
            pl.BlockSpec((1, D), lambda i: (0, 0)),
        ],
        out_specs=pl.BlockSpec((BE, D), lambda i: (i, 0)),
        out_shape=jax.ShapeDtypeStruct((E, D), jnp.float32),
    )(edge_attr, W1b, b1)


def _node_pre_body(x_ref, u_ref, w1a_ref, w2b_ref, w2c_ref, b2_ref, w4b_ref,
                   b4_ref, xw1_ref, xup_ref, uw4_ref):
    x = x_ref[...]
    u = u_ref[...]
    xw1_ref[...] = jnp.dot(x, w1a_ref[...], preferred_element_type=jnp.float32)
    xup_ref[...] = (
        jnp.dot(x, w2b_ref[...], preferred_element_type=jnp.float32)
        + jnp.dot(u, w2c_ref[...], preferred_element_type=jnp.float32)
        + b2_ref[...]
    )
    uw4_ref[...] = (
        jnp.dot(u, w4b_ref[...], preferred_element_type=jnp.float32)
        + b4_ref[...]
    )


def _node_pre(x, u, W1a, W2b, W2c, b2, W4b, b4):
    return pl.pallas_call(
        _node_pre_body,
        grid=(N // BN,),
        in_specs=[
            pl.BlockSpec((BN, D), lambda i: (i, 0)),
            pl.BlockSpec((BN, D), lambda i: (i, 0)),
            pl.BlockSpec((D, D), lambda i: (0, 0)),
            pl.BlockSpec((D, 4 * D), lambda i: (0, 0)),
            pl.BlockSpec((D, 4 * D), lambda i: (0, 0)),
            pl.BlockSpec((1, 4 * D), lambda i: (0, 0)),
            pl.BlockSpec((D, D), lambda i: (0, 0)),
            pl.BlockSpec((1, D), lambda i: (0, 0)),
        ],
        out_specs=[
            pl.BlockSpec((BN, D), lambda i: (i, 0)),
            pl.BlockSpec((BN, 4 * D), lambda i: (i, 0)),
            pl.BlockSpec((BN, D), lambda i: (i, 0)),
        ],
        out_shape=[
            jax.ShapeDtypeStruct((N, D), jnp.float32),
            jax.ShapeDtypeStruct((N, 4 * D), jnp.float32),
            jax.ShapeDtypeStruct((N, D), jnp.float32),
        ],
    )(x, u, W1a, W2b, W2c, b2, W4b, b4)


def _sc_agg_body(col_hbm, row_hbm, xw_hbm, ea_hbm, agg_hbm,
                 colbuf, rowbuf, eidl, rowl, lcoll, xg, eg, acc, cntb, nm_ref,
                 sem1, sem2):
    wid = lax.axis_index("s") * 2 + lax.axis_index("c")
    lo = wid * NPT
    hi = lo + NPT
    zero16 = jnp.zeros((16,), jnp.float32)
    zero16i = jnp.zeros((16,), jnp.int32)
    iota16 = lax.iota(jnp.int32, 16)

    def _z_acc(t, _):
        acc[pl.ds(t * 16, 16)] = zero16
        return 0
    lax.fori_loop(0, (NPT * D) // 16, _z_acc, 0)

    def _z_lists(t, _):
        eidl[pl.ds(t * 16, 16)] = zero16i
        rowl[pl.ds(t * 16, 16)] = zero16i
        return 0
    lax.fori_loop(0, LSZ // 16, _z_lists, 0)

    nm_ref[0] = 0

    def _flush():
        n = nm_ref[0]
        nit = (n + (GB - 1)) // GB

        def _gather_batch(k, _):
            off = k * GB
            cp1 = pltpu.async_copy(xw_hbm.at[rowl.at[pl.ds(off, GB)]], xg, sem1)
            cp2 = pltpu.async_copy(ea_hbm.at[eidl.at[pl.ds(off, GB)]], eg, sem2)
            cp1.wait()
            cp2.wait()

            def _row(r, _):
                @pl.when(off + r < n)
                def _():
                    lc = lcoll[pl.ds(off + r, 16)][0]
                    base = lc * D
                    for j in range(D // 16):
                        val = xg[r, pl.ds(16 * j, 16)] + eg[r, pl.ds(16 * j, 16)]
                        cur = acc[pl.ds(base + 16 * j, 16)]
                        acc[pl.ds(base + 16 * j, 16)] = jnp.maximum(cur, val)
                return 0

            lax.fori_loop(0, GB, _row, 0)
            return 0

        lax.fori_loop(0, nit, _gather_batch, 0)
        nm_ref[0] = 0

    def _chunk(c, _):
        pltpu.sync_copy(col_hbm.at[pl.ds(c * SCHUNK, SCHUNK)], colbuf)
        pltpu.sync_copy(row_hbm.at[pl.ds(c * SCHUNK, SCHUNK)], rowbuf)

        def _scan(t, _):
            v = colbuf[pl.ds(t * 16, 16)]
            r = rowbuf[pl.ds(t * 16, 16)]
            m = (v >= lo) & (v < hi)
            cntb[...] = plsc.all_reduce_population_count(m)
            cnt = cntb[pl.ds(0, 16)][0]
            nm = nm_ref[0]

            @pl.when(cnt > 0)
            def _():
                eids = c * SCHUNK + t * 16 + iota16
                plsc.store_compressed(lcoll.at[pl.ds(nm, 16)], v - lo, mask=m)
                plsc.store_compressed(rowl.at[pl.ds(nm, 16)], r, mask=m)
                plsc.store_compressed(eidl.at[pl.ds(nm, 16)], eids, mask=m)

            nm_ref[0] = nm + cnt

            @pl.when(nm + cnt >= CAP)
            def _():
                _flush()
            return 0

        lax.fori_loop(0, SCHUNK // 16, _scan, 0)
        return 0

    lax.fori_loop(0, NCH, _chunk, 0)
    _flush()

    pltpu.sync_copy(acc, agg_hbm.at[pl.ds(wid * NPT * D, NPT * D)])


def _sc_agg(col, row, xW1, eaW):
    mesh = plsc.VectorSubcoreMesh(core_axis_name="c", subcore_axis_name="s")
    f = functools.partial(
        pl.kernel,
        mesh=mesh,
        out_type=jax.ShapeDtypeStruct((NP * D,), jnp.float32),
        compiler_params=pltpu.CompilerParams(needs_layout_passes=False),
        scratch_types=[
            pltpu.VMEM((SCHUNK,), jnp.int32),
            pltpu.VMEM((SCHUNK,), jnp.int32),
            pltpu.VMEM((LSZ,), jnp.int32),
            pltpu.VMEM((LSZ,), jnp.int32),
            pltpu.VMEM((LSZ,), jnp.int32),
            pltpu.VMEM((GB, D), jnp.float32),
            pltpu.VMEM((GB, D), jnp.float32),
            pltpu.VMEM((NPT * D,), jnp.float32),
            pltpu.VMEM((16,), jnp.int32),
            pltpu.SMEM((1,), jnp.int32),
            pltpu.SemaphoreType.DMA,
            pltpu.SemaphoreType.DMA,
        ],
    )(_sc_agg_body)
    return f(col, row, xW1, eaW)


def _node_mlp_body(agg_ref, xup_ref, x_ref, batchv_ref, batchs_ref,
                   w2a_ref, w3_ref, b3_ref, x2_ref, sraw_ref, acc_ref):
    i = pl.program_id(0)
    neg = jnp.float32(-jnp.inf)

    @pl.when(i == 0)
    def _():
        acc_ref[...] = jnp.full((NG, D), neg, jnp.float32)

    r1 = jax.nn.relu(
        jnp.dot(agg_ref[...], w2a_ref[...], preferred_element_type=jnp.float32)
        + xup_ref[...]
    )
    h = jax.nn.sigmoid(
        jnp.dot(r1, w3_ref[...], preferred_element_type=jnp.float32)
        + b3_ref[...]
    )
    x2 = x_ref[...] + h
    x2_ref[...] = x2

    bv = batchv_ref[...]
    g_lo = batchs_ref[i * BN]
    g_hi = batchs_ref[i * BN + BN - 1]

    def _g(g, _):
        msk = bv == g
        m = jnp.max(jnp.where(msk, x2, neg), axis=0, keepdims=True)
        acc_ref[pl.ds(g, 1), :] = jnp.maximum(acc_ref[pl.ds(g, 1), :], m)
        return 0

    lax.fori_loop(g_lo, g_hi + 1, _g, 0, unroll=False)
    sraw_ref[...] = acc_ref[...]


def _node_mlp(agg, xup, x, batch2d, batch, W2a, W3, b3):
    return pl.pallas_call(
        _node_mlp_body,
        grid=(N // BN,),
        in_specs=[
            pl.BlockSpec((BN, D), lambda i: (i, 0)),
            pl.BlockSpec((BN, 4 * D), lambda i: (i, 0)),
            pl.BlockSpec((BN, D), lambda i: (i, 0)),
            pl.BlockSpec((BN, 1), lambda i: (i, 0)),
            pl.BlockSpec((N,), lambda i: (0,), memory_space=pltpu.SMEM),
            pl.BlockSpec((D, 4 * D), lambda i: (0, 0)),
            pl.BlockSpec((4 * D, D), lambda i: (0, 0)),
            pl.BlockSpec((1, D), lambda i: (0, 0)),
        ],
        out_specs=[
            pl.BlockSpec((BN, D), lambda i: (i, 0)),
            pl.BlockSpec((NG, D), lambda i: (0, 0)),
        ],
        out_shape=[
            jax.ShapeDtypeStruct((N, D), jnp.float32),
            jax.ShapeDtypeStruct((NG, D), jnp.float32),
        ],
        scratch_shapes=[pltpu.VMEM((NG, D), jnp.float32)],
    )(agg, xup, x, batch2d, batch, W2a, W3, b3)


def _glob_body(u_ref, uw4_ref, sraw_ref, w4a_ref, u2_ref):
    i = pl.program_id(0)
    s = sraw_ref[...]
    s = jnp.where(jnp.isinf(s), 0.0, s)
    t = jnp.dot(s, w4a_ref[...], preferred_element_type=jnp.float32)
    pad = jnp.concatenate([t, jnp.zeros((BN - NG, D), jnp.float32)], axis=0)
    addend = jnp.where(i == 0, pad, jnp.zeros_like(pad))
    u2_ref[...] = u_ref[...] + jax.nn.relu(uw4_ref[...] + addend)


def _glob(u, uw4, sraw, W4a):
    return pl.pallas_call(
        _glob_body,
        grid=(N // BN,),
        in_specs=[
            pl.BlockSpec((BN, D), lambda i: (i, 0)),
            pl.BlockSpec((BN, D), lambda i: (i, 0)),
            pl.BlockSpec((NG, D), lambda i: (0, 0)),
            pl.BlockSpec((D, D), lambda i: (0, 0)),
        ],
        out_specs=pl.BlockSpec((BN, D), lambda i: (i, 0)),
        out_shape=jax.ShapeDtypeStruct((N, D), jnp.float32),
    )(u, uw4, sraw, W4a)


def kernel(x, edge_index, edge_attr, u, batch, W1, b1, W2, b2, W3, b3, W4, b4):
    row = edge_index[0].astype(jnp.int32)
    col = edge_index[1].astype(jnp.int32)
    batch_i = batch.astype(jnp.int32)

    W1a = W1[:D]
    W1b = W1[D:]
    W2a = W2[:D]
    W2b = W2[D : 2 * D]
    W2c = W2[2 * D :]
    W4a = W4[:D]
    W4b = W4[D:]

    eaW = _edge_mm(edge_attr, W1b, b1.reshape(1, D))
    xW1, xup, uw4 = _node_pre(
        x, u, W1a, W2b, W2c, b2.reshape(1, 4 * D), W4b, b4.reshape(1, D)
    )

    agg_flat = _sc_agg(col, row, xW1, eaW)
    agg = agg_flat.reshape(NP, D)[:N]

    x2, sraw = _node_mlp(
        agg, xup, x, batch_i.reshape(N, 1), batch_i, W2a, W3, b3.reshape(1, D)
    )
    u2 = _glob(u, uw4, sraw, W4a)

    return (x2, edge_index, edge_attr, u2, batch)


# speedup vs baseline: 1.3202x; 1.1209x over previous
import functools

import jax
import jax.numpy as jnp
from jax import lax
from jax.experimental import pallas as pl
from jax.experimental.pallas import tpu as pltpu
from jax.experimental.pallas import tpu_sc as plsc

N = 10000
E = 160000
D = 256
NG = 64

NW = 32
NPT = 313
NP = NW * NPT
SCHUNK = 4000
NCH = E // SCHUNK
CAP = 1024
LSZ = 1088
GB = 32

BE = 1280
BN = 1000


def _edge_mm_body(ea_ref, w_ref, b_ref, out_ref):
    out_ref[...] = (
        jnp.dot(ea_ref[...], w_ref[...], preferred_element_type=jnp.float32)
        + b_ref[...]
    )


def _edge_mm(edge_attr, W1b, b1):
    return pl.pallas_call(
        _edge_mm_body,
        grid=(E // BE,),
        in_specs=[
            pl.BlockSpec((BE, D), lambda i: (i, 0)),
            pl.BlockSpec((D, D), lambda i: (0, 0)),
            pl.BlockSpec((1, D), lambda i: (0, 0)),
        ],
        out_specs=pl.BlockSpec((BE, D), lambda i: (i, 0)),
        out_shape=jax.ShapeDtypeStruct((E, D), jnp.float32),
    )(edge_attr, W1b, b1)


def _node_xw1_body(x_ref, w1a_ref, xw1_ref):
    xw1_ref[...] = jnp.dot(
        x_ref[...], w1a_ref[...], preferred_element_type=jnp.float32
    )


def _node_xw1(x, W1a):
    return pl.pallas_call(
        _node_xw1_body,
        grid=(N // BN,),
        in_specs=[
            pl.BlockSpec((BN, D), lambda i: (i, 0)),
            pl.BlockSpec((D, D), lambda i: (0, 0)),
        ],
        out_specs=pl.BlockSpec((BN, D), lambda i: (i, 0)),
        out_shape=jax.ShapeDtypeStruct((N, D), jnp.float32),
    )(x, W1a)


def _node_rest_body(x_ref, u_ref, w2b_ref, w2c_ref, b2_ref, w4b_ref,
                    b4_ref, xup_ref, uw4_ref):
    x = x_ref[...]
    u = u_ref[...]
    xup_ref[...] = (
        jnp.dot(x, w2b_ref[...], preferred_element_type=jnp.float32)
        + jnp.dot(u, w2c_ref[...], preferred_element_type=jnp.float32)
        + b2_ref[...]
    )
    uw4_ref[...] = (
        jnp.dot(u, w4b_ref[...], preferred_element_type=jnp.float32)
        + b4_ref[...]
    )


def _node_rest(x, u, W2b, W2c, b2, W4b, b4):
    return pl.pallas_call(
        _node_rest_body,
        grid=(N // BN,),
        in_specs=[
            pl.BlockSpec((BN, D), lambda i: (i, 0)),
            pl.BlockSpec((BN, D), lambda i: (i, 0)),
            pl.BlockSpec((D, 4 * D), lambda i: (0, 0)),
            pl.BlockSpec((D, 4 * D), lambda i: (0, 0)),
            pl.BlockSpec((1, 4 * D), lambda i: (0, 0)),
            pl.BlockSpec((D, D), lambda i: (0, 0)),
            pl.BlockSpec((1, D), lambda i: (0, 0)),
        ],
        out_specs=[
            pl.BlockSpec((BN, 4 * D), lambda i: (i, 0)),
            pl.BlockSpec((BN, D), lambda i: (i, 0)),
        ],
        out_shape=[
            jax.ShapeDtypeStruct((N, 4 * D), jnp.float32),
            jax.ShapeDtypeStruct((N, D), jnp.float32),
        ],
    )(x, u, W2b, W2c, b2, W4b, b4)


def _sc_agg_body(col_hbm, row_hbm, xw_hbm, ea_hbm, agg_hbm,
                 colbuf, rowbuf, eidl, rowl, lcoll, xga, ega, xgb, egb,
                 acc, cntb, nm_ref, semxa, semea, semxb, semeb):
    wid = lax.axis_index("s") * 2 + lax.axis_index("c")
    lo = wid * NPT
    hi = lo + NPT
    zero16 = jnp.zeros((16,), jnp.float32)
    zero16i = jnp.zeros((16,), jnp.int32)
    iota16 = lax.iota(jnp.int32, 16)

    def _z_acc(t, _):
        acc[pl.ds(t * 16, 16)] = zero16
        return 0
    lax.fori_loop(0, (NPT * D) // 16, _z_acc, 0)

    def _z_lists(t, _):
        eidl[pl.ds(t * 16, 16)] = zero16i
        rowl[pl.ds(t * 16, 16)] = zero16i
        return 0
    lax.fori_loop(0, LSZ // 16, _z_lists, 0)

    nm_ref[0] = 0

    def _issue(k, xg, eg, semx, seme):
        off = k * GB
        pltpu.async_copy(xw_hbm.at[rowl.at[pl.ds(off, GB)]], xg, semx)
        pltpu.async_copy(ea_hbm.at[eidl.at[pl.ds(off, GB)]], eg, seme)

    def _wait(xg, eg, semx, seme):
        pltpu.make_async_copy(xw_hbm.at[rowl.at[pl.ds(0, GB)]], xg, semx).wait()
        pltpu.make_async_copy(ea_hbm.at[eidl.at[pl.ds(0, GB)]], eg, seme).wait()

    def _process(k, n, xg, eg):
        off = k * GB

        def _row(r, _):
            @pl.when(off + r < n)
            def _():
                lc = lcoll[pl.ds(off + r, 16)][0]
                base = lc * D
                for j in range(D // 16):
                    val = xg[r, pl.ds(16 * j, 16)] + eg[r, pl.ds(16 * j, 16)]
                    cur = acc[pl.ds(base + 16 * j, 16)]
                    acc[pl.ds(base + 16 * j, 16)] = jnp.maximum(cur, val)
            return 0

        lax.fori_loop(0, GB, _row, 0)

    def _flush():
        n = nm_ref[0]
        nit = (n + (GB - 1)) // GB

        @pl.when(nit > 0)
        def _():
            _issue(0, xga, ega, semxa, semea)

        def _pair(p, _):
            k0 = 2 * p
            k1 = k0 + 1
            _wait(xga, ega, semxa, semea)

            @pl.when(k1 < nit)
            def _():
                _issue(k1, xgb, egb, semxb, semeb)

            _process(k0, n, xga, ega)

            @pl.when(k1 < nit)
            def _():
                _wait(xgb, egb, semxb, semeb)

                @pl.when(k1 + 1 < nit)
                def _():
                    _issue(k1 + 1, xga, ega, semxa, semea)

                _process(k1, n, xgb, egb)
            return 0

        lax.fori_loop(0, (nit + 1) // 2, _pair, 0)
        nm_ref[0] = 0

    def _chunk(c, _):
        pltpu.sync_copy(col_hbm.at[pl.ds(c * SCHUNK, SCHUNK)], colbuf)
        pltpu.sync_copy(row_hbm.at[pl.ds(c * SCHUNK, SCHUNK)], rowbuf)

        def _scan(t, _):
            v = colbuf[pl.ds(t * 16, 16)]
            r = rowbuf[pl.ds(t * 16, 16)]
            m = (v >= lo) & (v < hi)
            cntb[...] = plsc.all_reduce_population_count(m)
            cnt = cntb[pl.ds(0, 16)][0]
            nm = nm_ref[0]

            @pl.when(cnt > 0)
            def _():
                eids = c * SCHUNK + t * 16 + iota16
                plsc.store_compressed(lcoll.at[pl.ds(nm, 16)], v - lo, mask=m)
                plsc.store_compressed(rowl.at[pl.ds(nm, 16)], r, mask=m)
                plsc.store_compressed(eidl.at[pl.ds(nm, 16)], eids, mask=m)

            nm_ref[0] = nm + cnt

            @pl.when(nm + cnt >= CAP)
            def _():
                _flush()
            return 0

        lax.fori_loop(0, SCHUNK // 16, _scan, 0)
        return 0

    lax.fori_loop(0, NCH, _chunk, 0)
    _flush()

    pltpu.sync_copy(acc, agg_hbm.at[pl.ds(wid * NPT * D, NPT * D)])


def _sc_agg(col, row, xW1, eaW):
    mesh = plsc.VectorSubcoreMesh(core_axis_name="c", subcore_axis_name="s")
    f = functools.partial(
        pl.kernel,
        mesh=mesh,
        out_type=jax.ShapeDtypeStruct((NP * D,), jnp.float32),
        compiler_params=pltpu.CompilerParams(needs_layout_passes=False),
        scratch_types=[
            pltpu.VMEM((SCHUNK,), jnp.int32),
            pltpu.VMEM((SCHUNK,), jnp.int32),
            pltpu.VMEM((LSZ,), jnp.int32),
            pltpu.VMEM((LSZ,), jnp.int32),
            pltpu.VMEM((LSZ,), jnp.int32),
            pltpu.VMEM((GB, D), jnp.float32),
            pltpu.VMEM((GB, D), jnp.float32),
            pltpu.VMEM((GB, D), jnp.float32),
            pltpu.VMEM((GB, D), jnp.float32),
            pltpu.VMEM((NPT * D,), jnp.float32),
            pltpu.VMEM((16,), jnp.int32),
            pltpu.SMEM((1,), jnp.int32),
            pltpu.SemaphoreType.DMA,
            pltpu.SemaphoreType.DMA,
            pltpu.SemaphoreType.DMA,
            pltpu.SemaphoreType.DMA,
        ],
    )(_sc_agg_body)
    return f(col, row, xW1, eaW)


def _node_mlp_body(agg_ref, xup_ref, x_ref, batchv_ref, batchs_ref,
                   w2a_ref, w3_ref, b3_ref, x2_ref, sraw_ref, acc_ref):
    i = pl.program_id(0)
    neg = jnp.float32(-jnp.inf)

    @pl.when(i == 0)
    def _():
        acc_ref[...] = jnp.full((NG, D), neg, jnp.float32)

    r1 = jax.nn.relu(
        jnp.dot(agg_ref[...], w2a_ref[...], preferred_element_type=jnp.float32)
        + xup_ref[...]
    )
    h = jax.nn.sigmoid(
        jnp.dot(r1, w3_ref[...], preferred_element_type=jnp.float32)
        + b3_ref[...]
    )
    x2 = x_ref[...] + h
    x2_ref[...] = x2

    bv = batchv_ref[...]
    g_lo = batchs_ref[i * BN]
    g_hi = batchs_ref[i * BN + BN - 1]

    def _g(g, _):
        msk = bv == g
        m = jnp.max(jnp.where(msk, x2, neg), axis=0, keepdims=True)
        acc_ref[pl.ds(g, 1), :] = jnp.maximum(acc_ref[pl.ds(g, 1), :], m)
        return 0

    lax.fori_loop(g_lo, g_hi + 1, _g, 0, unroll=False)
    sraw_ref[...] = acc_ref[...]


def _node_mlp(agg, xup, x, batch2d, batch, W2a, W3, b3):
    return pl.pallas_call(
        _node_mlp_body,
        grid=(N // BN,),
        in_specs=[
            pl.BlockSpec((BN, D), lambda i: (i, 0)),
            pl.BlockSpec((BN, 4 * D), lambda i: (i, 0)),
            pl.BlockSpec((BN, D), lambda i: (i, 0)),
            pl.BlockSpec((BN, 1), lambda i: (i, 0)),
            pl.BlockSpec((N,), lambda i: (0,), memory_space=pltpu.SMEM),
            pl.BlockSpec((D, 4 * D), lambda i: (0, 0)),
            pl.BlockSpec((4 * D, D), lambda i: (0, 0)),
            pl.BlockSpec((1, D), lambda i: (0, 0)),
        ],
        out_specs=[
            pl.BlockSpec((BN, D), lambda i: (i, 0)),
            pl.BlockSpec((NG, D), lambda i: (0, 0)),
        ],
        out_shape=[
            jax.ShapeDtypeStruct((N, D), jnp.float32),
            jax.ShapeDtypeStruct((NG, D), jnp.float32),
        ],
        scratch_shapes=[pltpu.VMEM((NG, D), jnp.float32)],
    )(agg, xup, x, batch2d, batch, W2a, W3, b3)


def _glob_body(u_ref, uw4_ref, sraw_ref, w4a_ref, u2_ref):
    i = pl.program_id(0)
    s = sraw_ref[...]
    s = jnp.where(jnp.isinf(s), 0.0, s)
    t = jnp.dot(s, w4a_ref[...], preferred_element_type=jnp.float32)
    pad = jnp.concatenate([t, jnp.zeros((BN - NG, D), jnp.float32)], axis=0)
    addend = jnp.where(i == 0, pad, jnp.zeros_like(pad))
    u2_ref[...] = u_ref[...] + jax.nn.relu(uw4_ref[...] + addend)


def _glob(u, uw4, sraw, W4a):
    return pl.pallas_call(
        _glob_body,
        grid=(N // BN,),
        in_specs=[
            pl.BlockSpec((BN, D), lambda i: (i, 0)),
            pl.BlockSpec((BN, D), lambda i: (i, 0)),
            pl.BlockSpec((NG, D), lambda i: (0, 0)),
            pl.BlockSpec((D, D), lambda i: (0, 0)),
        ],
        out_specs=pl.BlockSpec((BN, D), lambda i: (i, 0)),
        out_shape=jax.ShapeDtypeStruct((N, D), jnp.float32),
    )(u, uw4, sraw, W4a)


def kernel(x, edge_index, edge_attr, u, batch, W1, b1, W2, b2, W3, b3, W4, b4):
    row = edge_index[0].astype(jnp.int32)
    col = edge_index[1].astype(jnp.int32)
    batch_i = batch.astype(jnp.int32)

    W1a = W1[:D]
    W1b = W1[D:]
    W2a = W2[:D]
    W2b = W2[D : 2 * D]
    W2c = W2[2 * D :]
    W4a = W4[:D]
    W4b = W4[D:]

    eaW = _edge_mm(edge_attr, W1b, b1.reshape(1, D))
    xW1 = _node_xw1(x, W1a)

    agg_flat = _sc_agg(col, row, xW1, eaW)
    xup, uw4 = _node_rest(
        x, u, W2b, W2c, b2.reshape(1, 4 * D), W4b, b4.reshape(1, D)
    )
    agg = agg_flat.reshape(NP, D)[:N]

    x2, sraw = _node_mlp(
        agg, xup, x, batch_i.reshape(N, 1), batch_i, W2a, W3, b3.reshape(1, D)
    )
    u2 = _glob(u, uw4, sraw, W4a)

    return (x2, edge_index, edge_attr, u2, batch)
